# baseline (speedup 1.0000x reference)
import jax
import jax.numpy as jnp
from jax.experimental import pallas as pl

_BM = 512
_BK = 2048


def _body(idx_ref, x_ref, o_ref):
    k0 = pl.program_id(1) * _BK
    idxv = idx_ref[...]
    cols = jax.lax.broadcasted_iota(jnp.int32, (_BK, 64), 0) + k0
    onehot = (cols == idxv).astype(jnp.float32)
    part = jnp.dot(x_ref[...], onehot, preferred_element_type=jnp.float32)

    @pl.when(pl.program_id(1) == 0)
    def _init():
        o_ref[...] = part

    @pl.when(pl.program_id(1) != 0)
    def _acc():
        o_ref[...] += part


def kernel(x, idx):
    m, k = x.shape
    n = idx.shape[0]
    idx2 = idx.reshape(1, n)
    grid = (m // _BM, k // _BK)
    return pl.pallas_call(
        _body,
        grid=grid,
        in_specs=[
            pl.BlockSpec((1, n), lambda i, j: (0, 0)),
            pl.BlockSpec((_BM, _BK), lambda i, j: (i, j)),
        ],
        out_specs=pl.BlockSpec((_BM, n), lambda i, j: (i, 0)),
        out_shape=jax.ShapeDtypeStruct((m, n), jnp.float32),
    )(idx2, x)


# speedup vs baseline: 1.2069x; 1.2069x over previous
import jax
import jax.numpy as jnp
from jax.experimental import pallas as pl

_BM = 512


def _body(idx_ref, x_ref, o_ref):
    idxv = idx_ref[...]
    cols = jax.lax.broadcasted_iota(jnp.int32, (4096, 64), 0)
    onehot = (cols == idxv).astype(jnp.float32)
    o_ref[...] = jnp.dot(x_ref[...], onehot, preferred_element_type=jnp.float32)


def kernel(x, idx):
    m, k = x.shape
    n = idx.shape[0]
    idx2 = idx.reshape(1, n)
    grid = (m // _BM,)
    return pl.pallas_call(
        _body,
        grid=grid,
        in_specs=[
            pl.BlockSpec((1, n), lambda i: (0, 0)),
            pl.BlockSpec((_BM, k), lambda i: (i, 0)),
        ],
        out_specs=pl.BlockSpec((_BM, n), lambda i: (i, 0)),
        out_shape=jax.ShapeDtypeStruct((m, n), jnp.float32),
    )(idx2, x)


# speedup vs baseline: 1.2088x; 1.0016x over previous
import jax
import jax.numpy as jnp
from jax.experimental import pallas as pl

_BM = 512


def _body(idx_ref, x_ref, o_ref):
    k, n = x_ref.shape[1], idx_ref.shape[1]
    idxv = idx_ref[...]
    cols = jax.lax.broadcasted_iota(jnp.int32, (k, n), 0)
    onehot = (cols == idxv).astype(jnp.float32)
    o_ref[...] = jnp.dot(x_ref[...], onehot, preferred_element_type=jnp.float32)


def kernel(x, idx):
    m, k = x.shape
    n = idx.shape[0]
    idx2 = idx.reshape(1, n)
    grid = (m // _BM,)
    return pl.pallas_call(
        _body,
        grid=grid,
        in_specs=[
            pl.BlockSpec((1, n), lambda i: (0, 0)),
            pl.BlockSpec((_BM, k), lambda i: (i, 0)),
        ],
        out_specs=pl.BlockSpec((_BM, n), lambda i: (i, 0)),
        out_shape=jax.ShapeDtypeStruct((m, n), jnp.float32),
    )(idx2, x)
